# Initial kernel scaffold; baseline (speedup 1.0000x reference)
import jax
import jax.numpy as jnp
from jax.experimental import pallas as pl


def kernel(batch_x, edge_index, glob_emb, W0, b0, W1, b1, attW1, attb1, attW2, linW, linb):
    raise NotImplementedError("write your pallas kernel here")


# speedup vs baseline: 22.0258x; 22.0258x over previous
import functools

import jax
import jax.numpy as jnp
from jax import lax
from jax.experimental import pallas as pl
from jax.experimental.pallas import tpu as pltpu
from jax.experimental.pallas import tpu_sc as plsc

N = 10000
NPAD = 10240
E = 320000
IN_CH = 128
HIDDEN = 64
LOC_OUT = 16
OUT_CH = 128

NCORE = 2
NSUB = 16
NW = NCORE * NSUB
EPC = 128
CPW = 80
EPAD = NW * CPW * EPC
NBUF = 4
NGRP = CPW // NBUF
RPT = NPAD // NSUB
DEGW = 16

_MESH = dict(core_axis_name="c", subcore_axis_name="s")
_SC_PARAMS = pltpu.CompilerParams(use_tc_tiling_on_sc=False)


def _wid():
    return lax.axis_index("s") * NCORE + lax.axis_index("c")


@functools.partial(
    pl.kernel,
    out_type=jax.ShapeDtypeStruct((NCORE, NPAD, DEGW), jnp.float32),
    mesh=plsc.VectorSubcoreMesh(**_MESH),
    compiler_params=_SC_PARAMS,
    scratch_types=[
        pltpu.VMEM_SHARED((NPAD, DEGW), jnp.float32),
        pltpu.VMEM((CPW, EPC), jnp.int32),
        pltpu.VMEM((EPC, DEGW), jnp.float32),
        pltpu.SemaphoreType.DMA,
        pltpu.SemaphoreType.DMA,
        pltpu.SemaphoreType.DMA,
        pltpu.SemaphoreType.DMA,
    ],
)
def _sc_deg(dst3, ones_hbm, zeros_hbm, out, dacc, dstidx, ones_v, *sems):
    c = lax.axis_index("c")
    s = lax.axis_index("s")
    w = _wid()
    pltpu.sync_copy(zeros_hbm.at[pl.ds(s * RPT, RPT)],
                    dacc.at[pl.ds(s * RPT, RPT)])
    pltpu.sync_copy(ones_hbm, ones_v)
    pltpu.sync_copy(dst3.at[w], dstidx)
    plsc.subcore_barrier()

    def grp(g, carry):
        for b in range(NBUF):
            j = g * NBUF + b

            @pl.when(g > 0)
            def _():
                pltpu.make_async_copy(ones_v, dacc.at[dstidx.at[j]],
                                      sems[b]).wait()

            pltpu.async_copy(ones_v, dacc.at[dstidx.at[j]], sems[b], add=True)
        return carry

    lax.fori_loop(0, NGRP, grp, 0)
    for b in range(NBUF):
        pltpu.make_async_copy(ones_v, dacc.at[dstidx.at[b]], sems[b]).wait()
    plsc.subcore_barrier()
    pltpu.sync_copy(dacc.at[pl.ds(s * RPT, RPT)],
                    out.at[c, pl.ds(s * RPT, RPT)])


def _make_sc_agg(width):
    @functools.partial(
        pl.kernel,
        out_type=jax.ShapeDtypeStruct((NCORE, NPAD, width), jnp.float32),
        mesh=plsc.VectorSubcoreMesh(**_MESH),
        compiler_params=_SC_PARAMS,
        scratch_types=[
            pltpu.VMEM_SHARED((NPAD, width), jnp.float32),
            pltpu.VMEM((CPW, EPC), jnp.int32),
            pltpu.VMEM((CPW, EPC), jnp.int32),
        ] + [pltpu.VMEM((EPC, width), jnp.float32) for _ in range(NBUF)]
          + [pltpu.SemaphoreType.DMA for _ in range(NBUF)],
    )
    def sc_agg(h_hbm, src3, dst3, zeros_hbm, out, acc, srcidx, dstidx, *rest):
        rows = rest[:NBUF]
        sems = rest[NBUF:]
        c = lax.axis_index("c")
        s = lax.axis_index("s")
        w = _wid()
        pltpu.sync_copy(zeros_hbm.at[pl.ds(s * RPT, RPT)],
                        acc.at[pl.ds(s * RPT, RPT)])
        pltpu.sync_copy(src3.at[w], srcidx)
        pltpu.sync_copy(dst3.at[w], dstidx)
        plsc.subcore_barrier()
        for b in range(NBUF):
            pltpu.async_copy(h_hbm.at[srcidx.at[b]], rows[b], sems[b])

        def grp(g, carry):
            for b in range(NBUF):
                j = g * NBUF + b
                pltpu.make_async_copy(h_hbm.at[srcidx.at[j]], rows[b],
                                      sems[b]).wait()
                pltpu.sync_copy(rows[b], acc.at[dstidx.at[j]], add=True)

                @pl.when(g < NGRP - 1)
                def _():
                    pltpu.async_copy(h_hbm.at[srcidx.at[j + NBUF]], rows[b],
                                     sems[b])
            return carry

        lax.fori_loop(0, NGRP, grp, 0)
        plsc.subcore_barrier()
        pltpu.sync_copy(acc.at[pl.ds(s * RPT, RPT)],
                        out.at[c, pl.ds(s * RPT, RPT)])

    return sc_agg


_sc_agg64 = _make_sc_agg(HIDDEN)
_sc_agg16 = _make_sc_agg(LOC_OUT)


BLK = 1024
GRID = NPAD // BLK


def _dis(deg_ref):
    d = deg_ref[...]
    deg = d[0][:, :1] + d[1][:, :1] + 1.0
    return lax.rsqrt(deg)


def _tc1_body(x_ref, w_ref, deg_ref, h_ref):
    dis = _dis(deg_ref)
    h = jnp.dot(x_ref[...], w_ref[...], preferred_element_type=jnp.float32)
    h_ref[...] = h * dis


def _tc1(x_p, W0, deg_parts):
    return pl.pallas_call(
        _tc1_body,
        grid=(GRID,),
        in_specs=[
            pl.BlockSpec((BLK, IN_CH), lambda i: (i, 0)),
            pl.BlockSpec((IN_CH, HIDDEN), lambda i: (0, 0)),
            pl.BlockSpec((NCORE, BLK, DEGW), lambda i: (0, i, 0)),
        ],
        out_specs=pl.BlockSpec((BLK, HIDDEN), lambda i: (i, 0)),
        out_shape=jax.ShapeDtypeStruct((NPAD, HIDDEN), jnp.float32),
    )(x_p, W0, deg_parts)


def _tc2_body(agg_ref, h_ref, deg_ref, b0_ref, w1_ref, o_ref):
    dis = _dis(deg_ref)
    a = agg_ref[0] + agg_ref[1] + h_ref[...]
    x1 = jax.nn.relu(a * dis + b0_ref[...])
    h1 = jnp.dot(x1, w1_ref[...], preferred_element_type=jnp.float32)
    o_ref[...] = h1 * dis


def _tc2(agg0, h0s, deg_parts, b0, W1):
    return pl.pallas_call(
        _tc2_body,
        grid=(GRID,),
        in_specs=[
            pl.BlockSpec((NCORE, BLK, HIDDEN), lambda i: (0, i, 0)),
            pl.BlockSpec((BLK, HIDDEN), lambda i: (i, 0)),
            pl.BlockSpec((NCORE, BLK, DEGW), lambda i: (0, i, 0)),
            pl.BlockSpec((1, HIDDEN), lambda i: (0, 0)),
            pl.BlockSpec((HIDDEN, LOC_OUT), lambda i: (0, 0)),
        ],
        out_specs=pl.BlockSpec((BLK, LOC_OUT), lambda i: (i, 0)),
        out_shape=jax.ShapeDtypeStruct((NPAD, LOC_OUT), jnp.float32),
    )(agg0, h0s, deg_parts, b0.reshape(1, HIDDEN), W1)


def _tc3_body(agg_ref, h_ref, deg_ref, b1_ref, g_ref, aw1_ref, ab1_ref,
              aw2_ref, lw_ref, lb_ref, o_ref):
    dis = _dis(deg_ref)
    x2 = (agg_ref[0] + agg_ref[1] + h_ref[...]) * dis + b1_ref[...]
    g = g_ref[...]
    aw1 = aw1_ref[...]
    ab1 = ab1_ref[...]
    aw2 = aw2_ref[...]
    t1 = jnp.tanh(jnp.dot(x2, aw1, preferred_element_type=jnp.float32) + ab1)
    t2 = jnp.tanh(jnp.dot(g, aw1, preferred_element_type=jnp.float32) + ab1)
    w1 = jnp.dot(t1, aw2, preferred_element_type=jnp.float32)
    w2 = jnp.dot(t2, aw2, preferred_element_type=jnp.float32)
    m = jnp.maximum(w1, w2)
    e1 = jnp.exp(w1 - m)
    e2 = jnp.exp(w2 - m)
    tot = e1 + e2
    emb = (e1 / tot) * x2 + (e2 / tot) * g
    o_ref[...] = (jnp.dot(emb, lw_ref[...], preferred_element_type=jnp.float32)
                  + lb_ref[...])


def _tc3(agg1, h1s, deg_parts, b1, glob_p, attW1, attb1, attW2, linW, linb):
    return pl.pallas_call(
        _tc3_body,
        grid=(GRID,),
        in_specs=[
            pl.BlockSpec((NCORE, BLK, LOC_OUT), lambda i: (0, i, 0)),
            pl.BlockSpec((BLK, LOC_OUT), lambda i: (i, 0)),
            pl.BlockSpec((NCORE, BLK, DEGW), lambda i: (0, i, 0)),
            pl.BlockSpec((1, LOC_OUT), lambda i: (0, 0)),
            pl.BlockSpec((BLK, LOC_OUT), lambda i: (i, 0)),
            pl.BlockSpec((LOC_OUT, 16), lambda i: (0, 0)),
            pl.BlockSpec((1, 16), lambda i: (0, 0)),
            pl.BlockSpec((16, 1), lambda i: (0, 0)),
            pl.BlockSpec((LOC_OUT, OUT_CH), lambda i: (0, 0)),
            pl.BlockSpec((1, OUT_CH), lambda i: (0, 0)),
        ],
        out_specs=pl.BlockSpec((BLK, OUT_CH), lambda i: (i, 0)),
        out_shape=jax.ShapeDtypeStruct((NPAD, OUT_CH), jnp.float32),
    )(agg1, h1s, deg_parts, b1.reshape(1, LOC_OUT), glob_p, attW1,
      attb1.reshape(1, 16), attW2, linW, linb.reshape(1, OUT_CH))


def kernel(batch_x, edge_index, glob_emb, W0, b0, W1, b1, attW1, attb1,
           attW2, linW, linb):
    src = edge_index[0]
    dst = edge_index[1]
    pad_e = EPAD - E
    src_p = jnp.concatenate([src, jnp.zeros((pad_e,), jnp.int32)])
    dst_p = jnp.concatenate([dst, jnp.full((pad_e,), N, jnp.int32)])
    src3 = src_p.reshape(NW, CPW, EPC)
    dst3 = dst_p.reshape(NW, CPW, EPC)

    x_p = jnp.pad(batch_x, ((0, NPAD - N), (0, 0)))
    glob_p = jnp.pad(glob_emb, ((0, NPAD - N), (0, 0)))
    zeros64 = jnp.zeros((NPAD, HIDDEN), jnp.float32)
    zeros16 = jnp.zeros((NPAD, LOC_OUT), jnp.float32)
    ones_hbm = jnp.ones((EPC, DEGW), jnp.float32)

    deg_parts = _sc_deg(dst3, ones_hbm, zeros16)
    h0s = _tc1(x_p, W0, deg_parts)
    agg0 = _sc_agg64(h0s, src3, dst3, zeros64)
    h1s = _tc2(agg0, h0s, deg_parts, b0, W1)
    agg1 = _sc_agg16(h1s, src3, dst3, zeros16)
    res_p = _tc3(agg1, h1s, deg_parts, b1, glob_p, attW1, attb1, attW2,
                 linW, linb)
    return res_p[:N]
